# Initial kernel scaffold; baseline (speedup 1.0000x reference)
import jax
import jax.numpy as jnp
from jax.experimental import pallas as pl


def kernel(features, mesh_feats, g2m_attr, mm_attr, m2g_attr, params, g2m_src, g2m_dst, mm_src, mm_dst, m2g_src, m2g_dst):
    raise NotImplementedError("write your pallas kernel here")


# speedup vs baseline: 1.0126x; 1.0126x over previous
import functools

import jax
import jax.numpy as jnp
from jax import lax
from jax.experimental import pallas as pl
from jax.experimental.pallas import tpu as pltpu

H = 256
F32 = jnp.float32


def _ln(h):
    m = jnp.mean(h, axis=-1, keepdims=True)
    c = h - m
    v = jnp.mean(c * c, axis=-1, keepdims=True)
    return c * lax.rsqrt(v + 1e-5)


def _dot(a, b):
    return jnp.dot(a, b, preferred_element_type=F32)


def _embed3_body(x_ref, w1, b1, w2, b2, w3, b3, o_ref):
    h = jnp.maximum(_dot(x_ref[...], w1[...]) + b1[...], 0.0)
    h = jnp.maximum(_dot(h, w2[...]) + b2[...], 0.0)
    h = _dot(h, w3[...]) + b3[...]
    o_ref[...] = _ln(h)


def _embed3(x, ps, bm):
    (w1, b1), (w2, b2), (w3, b3) = ps
    M, K = x.shape
    w1 = jnp.pad(w1, ((0, K - w1.shape[0]), (0, 0)))
    grid = M // bm
    wspec = lambda r, c: pl.BlockSpec((r, c), lambda i: (0, 0))
    return pl.pallas_call(
        _embed3_body,
        grid=(grid,),
        in_specs=[
            pl.BlockSpec((bm, K), lambda i: (i, 0)),
            wspec(K, H), wspec(1, H), wspec(H, H), wspec(1, H), wspec(H, H), wspec(1, H),
        ],
        out_specs=pl.BlockSpec((bm, H), lambda i: (i, 0)),
        out_shape=jax.ShapeDtypeStruct((M, H), F32),
        compiler_params=pltpu.CompilerParams(dimension_semantics=("arbitrary",)),
    )(x, w1, b1.reshape(1, H), w2, b2.reshape(1, H), w3, b3.reshape(1, H))


def _edge3_body(e_ref, gb_ref, gc_ref, w1a, b1, w2, b2, w3, b3, o_ref):
    e = e_ref[...]
    h = jnp.maximum(_dot(e, w1a[...]) + gb_ref[...] + gc_ref[...] + b1[...], 0.0)
    h = jnp.maximum(_dot(h, w2[...]) + b2[...], 0.0)
    h = _dot(h, w3[...]) + b3[...]
    o_ref[...] = e + _ln(h)


def _edge3(e, gb, gc, w1a, b1, w2, b2, w3, b3, bm):
    M = e.shape[0]
    grid = M // bm
    dspec = pl.BlockSpec((bm, H), lambda i: (i, 0))
    wspec = lambda r, c: pl.BlockSpec((r, c), lambda i: (0, 0))
    return pl.pallas_call(
        _edge3_body,
        grid=(grid,),
        in_specs=[dspec, dspec, dspec,
                  wspec(H, H), wspec(1, H), wspec(H, H), wspec(1, H), wspec(H, H), wspec(1, H)],
        out_specs=dspec,
        out_shape=jax.ShapeDtypeStruct((M, H), F32),
        compiler_params=pltpu.CompilerParams(dimension_semantics=("arbitrary",)),
    )(e, gb, gc, w1a, b1.reshape(1, H), w2, b2.reshape(1, H), w3, b3.reshape(1, H))


def _node3_body(nproj, x_ref, a0_ref, a1_ref, v1a, v1b, b1, v2, b2, v3, b3, p1, p2, o_ref, pb_ref, pc_ref):
    x = x_ref[...]
    agg = a0_ref[...] + a1_ref[...]
    h = jnp.maximum(_dot(x, v1a[...]) + _dot(agg, v1b[...]) + b1[...], 0.0)
    h = jnp.maximum(_dot(h, v2[...]) + b2[...], 0.0)
    h = _dot(h, v3[...]) + b3[...]
    xn = x + _ln(h)
    o_ref[...] = xn
    if nproj:
        pb_ref[...] = _dot(xn, p1[...])
        pc_ref[...] = _dot(xn, p2[...])


def _node3_noproj_body(x_ref, a0_ref, a1_ref, v1a, v1b, b1, v2, b2, v3, b3, o_ref):
    x = x_ref[...]
    agg = a0_ref[...] + a1_ref[...]
    h = jnp.maximum(_dot(x, v1a[...]) + _dot(agg, v1b[...]) + b1[...], 0.0)
    h = jnp.maximum(_dot(h, v2[...]) + b2[...], 0.0)
    h = _dot(h, v3[...]) + b3[...]
    o_ref[...] = x + _ln(h)


def _node3(x, a0, a1, v1a, b1, v1b, v2, b2, v3, b3, proj, bm):
    M = x.shape[0]
    grid = M // bm
    dspec = pl.BlockSpec((bm, H), lambda i: (i, 0))
    wspec = lambda: pl.BlockSpec((H, H), lambda i: (0, 0))
    bspec = lambda: pl.BlockSpec((1, H), lambda i: (0, 0))
    if proj is None:
        return pl.pallas_call(
            _node3_noproj_body,
            grid=(grid,),
            in_specs=[dspec, dspec, dspec,
                      wspec(), wspec(), bspec(), wspec(), bspec(), wspec(), bspec()],
            out_specs=dspec,
            out_shape=jax.ShapeDtypeStruct((M, H), F32),
            compiler_params=pltpu.CompilerParams(dimension_semantics=("arbitrary",)),
        )(x, a0, a1, v1a, v1b, b1.reshape(1, H), v2, b2.reshape(1, H), v3, b3.reshape(1, H))
    p1, p2 = proj
    return pl.pallas_call(
        functools.partial(_node3_body, True),
        grid=(grid,),
        in_specs=[dspec, dspec, dspec,
                  wspec(), wspec(), bspec(), wspec(), bspec(), wspec(), bspec(),
                  wspec(), wspec()],
        out_specs=(dspec, dspec, dspec),
        out_shape=(jax.ShapeDtypeStruct((M, H), F32),
                   jax.ShapeDtypeStruct((M, H), F32),
                   jax.ShapeDtypeStruct((M, H), F32)),
        compiler_params=pltpu.CompilerParams(dimension_semantics=("arbitrary",)),
    )(x, a0, a1, v1a, v1b, b1.reshape(1, H), v2, b2.reshape(1, H), v3, b3.reshape(1, H), p1, p2)


def _proj_body(x_ref, w_ref, o_ref):
    o_ref[...] = _dot(x_ref[...], w_ref[...])


def _proj(x, w, bm):
    M = x.shape[0]
    N = w.shape[1]
    return pl.pallas_call(
        _proj_body,
        grid=(M // bm,),
        in_specs=[pl.BlockSpec((bm, H), lambda i: (i, 0)),
                  pl.BlockSpec((H, N), lambda i: (0, 0))],
        out_specs=pl.BlockSpec((bm, N), lambda i: (i, 0)),
        out_shape=jax.ShapeDtypeStruct((M, N), F32),
        compiler_params=pltpu.CompilerParams(dimension_semantics=("arbitrary",)),
    )(x, w)


def _out3_body(x_ref, w1, b1, w2, b2, w3, b3, o_ref):
    h = jnp.maximum(_dot(x_ref[...], w1[...]) + b1[...], 0.0)
    h = jnp.maximum(_dot(h, w2[...]) + b2[...], 0.0)
    o_ref[...] = _dot(h, w3[...]) + b3[...]


def _out3(x, ps):
    (w1, b1), (w2, b2), (w3, b3) = ps
    M = x.shape[0]
    N = 128
    w3p = jnp.pad(w3, ((0, 0), (0, N - w3.shape[1])))
    b3p = jnp.pad(b3, (0, N - b3.shape[0])).reshape(1, N)
    wspec = lambda r, c: pl.BlockSpec((r, c), lambda i: (0, 0))
    return pl.pallas_call(
        _out3_body,
        grid=(1,),
        in_specs=[pl.BlockSpec((M, H), lambda i: (0, 0)),
                  wspec(H, H), wspec(1, H), wspec(H, H), wspec(1, H), wspec(H, N), wspec(1, N)],
        out_specs=pl.BlockSpec((M, N), lambda i: (0, 0)),
        out_shape=jax.ShapeDtypeStruct((M, N), F32),
    )(x, w1, b1.reshape(1, H), w2, b2.reshape(1, H), w3p, b3p)


def _gather(table, idx):
    return jnp.take(table, idx, axis=0)


def _scatter2(vals, idx, n):
    s = jax.ops.segment_sum(vals, idx, num_segments=n)
    return s, jnp.zeros_like(s)


def _padr(x, n, k=None):
    pc = 0 if k is None else k - x.shape[1]
    return jnp.pad(x, ((0, n - x.shape[0]), (0, pc)))


def _padi(idx, n, fill):
    return jnp.pad(idx, (0, n - idx.shape[0]), constant_values=fill).astype(jnp.int32)


def _split_edge_w(ps):
    (w1, b1), (w2, b2), (w3, b3) = ps
    return (w1[:H], w1[H:2 * H], w1[2 * H:], b1, w2, b2, w3, b3)


def _split_node_w(ps):
    (w1, b1), (w2, b2), (w3, b3) = ps
    return (w1[:H], w1[H:], b1, w2, b2, w3, b3)


def kernel(features, mesh_feats, g2m_attr, mm_attr, m2g_attr, params, g2m_src,
           g2m_dst, mm_src, mm_dst, m2g_src, m2g_dst):
    p = params
    NGp, NMp = 304, 5888
    EGp, EMp, EDp = 1024, 36864, 1024

    feat = _padr(features[0], NGp, 80)

    gx = _embed3(feat, p['grid_embed'], bm=NGp)
    mx = _embed3(_padr(mesh_feats, NMp, 8), p['mesh_embed'], bm=736)
    ge = _embed3(_padr(g2m_attr, EGp, 8), p['g2m_edge_embed'], bm=512)
    me = _embed3(_padr(mm_attr, EMp, 8), p['mm_edge_embed'], bm=512)
    de = _embed3(_padr(m2g_attr, EDp, 8), p['m2g_edge_embed'], bm=512)

    eWa, eWb, eWc, eb1, eW2, eb2, eW3, eb3 = _split_edge_w(p['enc_edge'])
    dWa, dWb, dWc, db1, dW2, db2, dW3, db3 = _split_edge_w(p['dec_edge'])
    pe = [_split_edge_w(ps) for ps in p['proc_edge']]
    pn = [_split_node_w(ps) for ps in p['proc_node']]

    g2m_srcp = _padi(g2m_src, EGp, NGp - 1)
    g2m_dstp = _padi(g2m_dst, EGp, NMp - 1)
    mm_srcp = _padi(mm_src, EMp, NMp - 1)
    mm_dstp = _padi(mm_dst, EMp, NMp - 1)
    m2g_srcp = _padi(m2g_src, EDp, NMp - 1)
    m2g_dstp = _padi(m2g_dst, EDp, NGp - 1)

    gP = _proj(gx, jnp.concatenate([eWb, dWc], axis=1), bm=NGp)
    Pb_enc, Pc_dec = gP[:, :H], gP[:, H:]
    Pc_enc = _proj(mx, eWc, bm=736)

    gb = _gather(Pb_enc, g2m_srcp)
    gc = _gather(Pc_enc, g2m_dstp)
    ge = _edge3(ge, gb, gc, eWa, eb1, eW2, eb2, eW3, eb3, bm=512)
    a0, a1 = _scatter2(ge, g2m_dstp, NMp)
    v1a, v1b, b1, v2, b2, v3, b3 = _split_node_w(p['enc_node'])
    mx, Pb, Pc = _node3(mx, a0, a1, v1a, b1, v1b, v2, b2, v3, b3,
                        proj=(pe[0][1], pe[0][2]), bm=736)

    for i in range(9):
        wa, _, _, b1e, w2e, b2e, w3e, b3e = pe[i]
        gb = _gather(Pb, mm_srcp)
        gc = _gather(Pc, mm_dstp)
        me = _edge3(me, gb, gc, wa, b1e, w2e, b2e, w3e, b3e, bm=512)
        a0, a1 = _scatter2(me, mm_dstp, NMp)
        v1a, v1b, b1, v2, b2, v3, b3 = pn[i]
        nxt = (pe[i + 1][1], pe[i + 1][2]) if i < 8 else (dWb, dWb)
        mx, Pb, Pc = _node3(mx, a0, a1, v1a, b1, v1b, v2, b2, v3, b3,
                            proj=nxt, bm=736)

    gb = _gather(Pb, m2g_srcp)
    gc = _gather(Pc_dec, m2g_dstp)
    de = _edge3(de, gb, gc, dWa, db1, dW2, db2, dW3, db3, bm=512)
    a0, a1 = _scatter2(de, m2g_dstp, NGp)
    v1a, v1b, b1, v2, b2, v3, b3 = _split_node_w(p['dec_node'])
    gx = _node3(gx, a0, a1, v1a, b1, v1b, v2, b2, v3, b3, proj=None, bm=NGp)

    out = _out3(gx, p['out'])
    return out[:288, :78][None]


# speedup vs baseline: 1.1730x; 1.1584x over previous
import functools

import jax
import jax.numpy as jnp
from jax import lax
from jax.experimental import pallas as pl
from jax.experimental.pallas import tpu as pltpu
from jax.experimental.pallas import tpu_sc as plsc

H = 256
F32 = jnp.float32


def _ln(h):
    m = jnp.mean(h, axis=-1, keepdims=True)
    c = h - m
    v = jnp.mean(c * c, axis=-1, keepdims=True)
    return c * lax.rsqrt(v + 1e-5)


def _dot(a, b):
    return jnp.dot(a, b, preferred_element_type=F32)


def _embed3_body(x_ref, w1, b1, w2, b2, w3, b3, o_ref):
    h = jnp.maximum(_dot(x_ref[...], w1[...]) + b1[...], 0.0)
    h = jnp.maximum(_dot(h, w2[...]) + b2[...], 0.0)
    h = _dot(h, w3[...]) + b3[...]
    o_ref[...] = _ln(h)


def _embed3(x, ps, bm):
    (w1, b1), (w2, b2), (w3, b3) = ps
    M, K = x.shape
    w1 = jnp.pad(w1, ((0, K - w1.shape[0]), (0, 0)))
    grid = M // bm
    wspec = lambda r, c: pl.BlockSpec((r, c), lambda i: (0, 0))
    return pl.pallas_call(
        _embed3_body,
        grid=(grid,),
        in_specs=[
            pl.BlockSpec((bm, K), lambda i: (i, 0)),
            wspec(K, H), wspec(1, H), wspec(H, H), wspec(1, H), wspec(H, H), wspec(1, H),
        ],
        out_specs=pl.BlockSpec((bm, H), lambda i: (i, 0)),
        out_shape=jax.ShapeDtypeStruct((M, H), F32),
        compiler_params=pltpu.CompilerParams(dimension_semantics=("arbitrary",)),
    )(x, w1, b1.reshape(1, H), w2, b2.reshape(1, H), w3, b3.reshape(1, H))


def _edge3_body(e_ref, gb_ref, gc_ref, w1a, b1, w2, b2, w3, b3, o_ref):
    e = e_ref[...]
    h = jnp.maximum(_dot(e, w1a[...]) + gb_ref[...] + gc_ref[...] + b1[...], 0.0)
    h = jnp.maximum(_dot(h, w2[...]) + b2[...], 0.0)
    h = _dot(h, w3[...]) + b3[...]
    o_ref[...] = e + _ln(h)


def _edge3(e, gb, gc, w1a, b1, w2, b2, w3, b3, bm):
    M = e.shape[0]
    grid = M // bm
    dspec = pl.BlockSpec((bm, H), lambda i: (i, 0))
    wspec = lambda r, c: pl.BlockSpec((r, c), lambda i: (0, 0))
    return pl.pallas_call(
        _edge3_body,
        grid=(grid,),
        in_specs=[dspec, dspec, dspec,
                  wspec(H, H), wspec(1, H), wspec(H, H), wspec(1, H), wspec(H, H), wspec(1, H)],
        out_specs=dspec,
        out_shape=jax.ShapeDtypeStruct((M, H), F32),
        compiler_params=pltpu.CompilerParams(dimension_semantics=("arbitrary",)),
    )(e, gb, gc, w1a, b1.reshape(1, H), w2, b2.reshape(1, H), w3, b3.reshape(1, H))


def _node3_body(nproj, x_ref, a0_ref, a1_ref, v1a, v1b, b1, v2, b2, v3, b3, p1, p2, o_ref, pb_ref, pc_ref):
    x = x_ref[...]
    agg = a0_ref[...] + a1_ref[...]
    h = jnp.maximum(_dot(x, v1a[...]) + _dot(agg, v1b[...]) + b1[...], 0.0)
    h = jnp.maximum(_dot(h, v2[...]) + b2[...], 0.0)
    h = _dot(h, v3[...]) + b3[...]
    xn = x + _ln(h)
    o_ref[...] = xn
    if nproj:
        pb_ref[...] = _dot(xn, p1[...])
        pc_ref[...] = _dot(xn, p2[...])


def _node3_noproj_body(x_ref, a0_ref, a1_ref, v1a, v1b, b1, v2, b2, v3, b3, o_ref):
    x = x_ref[...]
    agg = a0_ref[...] + a1_ref[...]
    h = jnp.maximum(_dot(x, v1a[...]) + _dot(agg, v1b[...]) + b1[...], 0.0)
    h = jnp.maximum(_dot(h, v2[...]) + b2[...], 0.0)
    h = _dot(h, v3[...]) + b3[...]
    o_ref[...] = x + _ln(h)


def _node3(x, a0, a1, v1a, b1, v1b, v2, b2, v3, b3, proj, bm):
    M = x.shape[0]
    grid = M // bm
    dspec = pl.BlockSpec((bm, H), lambda i: (i, 0))
    wspec = lambda: pl.BlockSpec((H, H), lambda i: (0, 0))
    bspec = lambda: pl.BlockSpec((1, H), lambda i: (0, 0))
    if proj is None:
        return pl.pallas_call(
            _node3_noproj_body,
            grid=(grid,),
            in_specs=[dspec, dspec, dspec,
                      wspec(), wspec(), bspec(), wspec(), bspec(), wspec(), bspec()],
            out_specs=dspec,
            out_shape=jax.ShapeDtypeStruct((M, H), F32),
            compiler_params=pltpu.CompilerParams(dimension_semantics=("arbitrary",)),
        )(x, a0, a1, v1a, v1b, b1.reshape(1, H), v2, b2.reshape(1, H), v3, b3.reshape(1, H))
    p1, p2 = proj
    return pl.pallas_call(
        functools.partial(_node3_body, True),
        grid=(grid,),
        in_specs=[dspec, dspec, dspec,
                  wspec(), wspec(), bspec(), wspec(), bspec(), wspec(), bspec(),
                  wspec(), wspec()],
        out_specs=(dspec, dspec, dspec),
        out_shape=(jax.ShapeDtypeStruct((M, H), F32),
                   jax.ShapeDtypeStruct((M, H), F32),
                   jax.ShapeDtypeStruct((M, H), F32)),
        compiler_params=pltpu.CompilerParams(dimension_semantics=("arbitrary",)),
    )(x, a0, a1, v1a, v1b, b1.reshape(1, H), v2, b2.reshape(1, H), v3, b3.reshape(1, H), p1, p2)


def _proj_body(x_ref, w_ref, o_ref):
    o_ref[...] = _dot(x_ref[...], w_ref[...])


def _proj(x, w, bm):
    M = x.shape[0]
    N = w.shape[1]
    return pl.pallas_call(
        _proj_body,
        grid=(M // bm,),
        in_specs=[pl.BlockSpec((bm, H), lambda i: (i, 0)),
                  pl.BlockSpec((H, N), lambda i: (0, 0))],
        out_specs=pl.BlockSpec((bm, N), lambda i: (i, 0)),
        out_shape=jax.ShapeDtypeStruct((M, N), F32),
        compiler_params=pltpu.CompilerParams(dimension_semantics=("arbitrary",)),
    )(x, w)


def _out3_body(x_ref, w1, b1, w2, b2, w3, b3, o_ref):
    h = jnp.maximum(_dot(x_ref[...], w1[...]) + b1[...], 0.0)
    h = jnp.maximum(_dot(h, w2[...]) + b2[...], 0.0)
    o_ref[...] = _dot(h, w3[...]) + b3[...]


def _out3(x, ps):
    (w1, b1), (w2, b2), (w3, b3) = ps
    M = x.shape[0]
    N = 128
    w3p = jnp.pad(w3, ((0, 0), (0, N - w3.shape[1])))
    b3p = jnp.pad(b3, (0, N - b3.shape[0])).reshape(1, N)
    wspec = lambda r, c: pl.BlockSpec((r, c), lambda i: (0, 0))
    return pl.pallas_call(
        _out3_body,
        grid=(1,),
        in_specs=[pl.BlockSpec((M, H), lambda i: (0, 0)),
                  wspec(H, H), wspec(1, H), wspec(H, H), wspec(1, H), wspec(H, N), wspec(1, N)],
        out_specs=pl.BlockSpec((M, N), lambda i: (0, 0)),
        out_shape=jax.ShapeDtypeStruct((M, N), F32),
    )(x, w1, b1.reshape(1, H), w2, b2.reshape(1, H), w3p, b3p)


_NC, _NS = 2, 16
_NW = _NC * _NS


def _sc_gather2(tb, ib, tc, ic, nchunks):
    Ep = ib.shape[0]
    ch = Ep // (_NW * nchunks)
    mesh = plsc.VectorSubcoreMesh(core_axis_name="c", subcore_axis_name="s")

    @functools.partial(
        pl.kernel, mesh=mesh,
        out_type=(jax.ShapeDtypeStruct((Ep, H), F32),
                  jax.ShapeDtypeStruct((Ep, H), F32)),
        scratch_types=[pltpu.VMEM((ch,), jnp.int32),
                       pltpu.VMEM((ch, H), F32),
                       pltpu.SemaphoreType.DMA],
    )
    def k(tb_h, ib_h, tc_h, ic_h, ob_h, oc_h, idx_v, rows_v, sem):
        wid = lax.axis_index("s") * _NC + lax.axis_index("c")
        for t_h, i_h, o_h in ((tb_h, ib_h, ob_h), (tc_h, ic_h, oc_h)):
            for j in range(nchunks):
                base = wid * (ch * nchunks) + j * ch
                pltpu.sync_copy(i_h.at[pl.ds(base, ch)], idx_v)
                pltpu.async_copy(t_h.at[idx_v], rows_v, sem).wait()
                pltpu.sync_copy(rows_v, o_h.at[pl.ds(base, ch)])

    return k(tb, ib, tc, ic)


def _sc_scatter2(vals, idx, n, nchunks, zeros):
    Ep = idx.shape[0]
    ch = Ep // (_NW * nchunks)
    rpt = n // _NS
    mesh = plsc.VectorSubcoreMesh(core_axis_name="c", subcore_axis_name="s")

    @functools.partial(
        pl.kernel, mesh=mesh,
        out_type=jax.ShapeDtypeStruct((_NC, n, H), F32),
        scratch_types=[pltpu.VMEM((128, H), F32),
                       pltpu.VMEM((ch,), jnp.int32),
                       pltpu.VMEM((ch, H), F32),
                       pltpu.SemaphoreType.DMA],
    )
    def k(v_h, i_h, z_h, o_h, zrows, idx_v, rows_v, sem):
        c = lax.axis_index("c")
        s = lax.axis_index("s")
        wid = s * _NC + c
        pltpu.sync_copy(z_h, zrows)
        r0 = s * rpt
        left = rpt
        while left > 0:
            w = min(128, left)
            pltpu.sync_copy(zrows.at[pl.ds(0, w)],
                            o_h.at[c, pl.ds(r0 + rpt - left, w)])
            left -= w
        plsc.subcore_barrier()
        for j in range(nchunks):
            base = wid * (ch * nchunks) + j * ch
            pltpu.sync_copy(i_h.at[pl.ds(base, ch)], idx_v)
            pltpu.async_copy(v_h.at[pl.ds(base, ch)], rows_v, sem).wait()
            pltpu.async_copy(rows_v, o_h.at[c].at[idx_v], sem, add=True).wait()

    out = k(vals, idx, zeros)
    return out[0], out[1]


def _jnp_scatter2(vals, idx, n, nchunks, zeros):
    s = jax.ops.segment_sum(vals, idx, num_segments=n)
    return s, jnp.zeros_like(s)


def _padr(x, n, k=None):
    pc = 0 if k is None else k - x.shape[1]
    return jnp.pad(x, ((0, n - x.shape[0]), (0, pc)))


def _padi(idx, n, fill):
    return jnp.pad(idx, (0, n - idx.shape[0]), constant_values=fill).astype(jnp.int32)


def _split_edge_w(ps):
    (w1, b1), (w2, b2), (w3, b3) = ps
    return (w1[:H], w1[H:2 * H], w1[2 * H:], b1, w2, b2, w3, b3)


def _split_node_w(ps):
    (w1, b1), (w2, b2), (w3, b3) = ps
    return (w1[:H], w1[H:], b1, w2, b2, w3, b3)


def kernel(features, mesh_feats, g2m_attr, mm_attr, m2g_attr, params, g2m_src,
           g2m_dst, mm_src, mm_dst, m2g_src, m2g_dst):
    p = params
    NGp, NMp = 384, 5888
    EGp, EMp, EDp = 1024, 36864, 1024

    feat = _padr(features[0], NGp, 80)

    gx = _embed3(feat, p['grid_embed'], bm=NGp)
    mx = _embed3(_padr(mesh_feats, NMp, 8), p['mesh_embed'], bm=736)
    ge = _embed3(_padr(g2m_attr, EGp, 8), p['g2m_edge_embed'], bm=512)
    me = _embed3(_padr(mm_attr, EMp, 8), p['mm_edge_embed'], bm=512)
    de = _embed3(_padr(m2g_attr, EDp, 8), p['m2g_edge_embed'], bm=512)

    eWa, eWb, eWc, eb1, eW2, eb2, eW3, eb3 = _split_edge_w(p['enc_edge'])
    dWa, dWb, dWc, db1, dW2, db2, dW3, db3 = _split_edge_w(p['dec_edge'])
    pe = [_split_edge_w(ps) for ps in p['proc_edge']]
    pn = [_split_node_w(ps) for ps in p['proc_node']]

    g2m_srcp = _padi(g2m_src, EGp, NGp - 1)
    g2m_dstp = _padi(g2m_dst, EGp, NMp - 1)
    mm_srcp = _padi(mm_src, EMp, NMp - 1)
    mm_dstp = _padi(mm_dst, EMp, NMp - 1)
    m2g_srcp = _padi(m2g_src, EDp, NMp - 1)
    m2g_dstp = _padi(m2g_dst, EDp, NGp - 1)

    gP = _proj(gx, jnp.concatenate([eWb, dWc], axis=1), bm=NGp)
    Pb_enc, Pc_dec = gP[:, :H], gP[:, H:]
    Pc_enc = _proj(mx, eWc, bm=736)

    zeros128 = jnp.zeros((128, H), F32)

    gb, gc = _sc_gather2(Pb_enc, g2m_srcp, Pc_enc, g2m_dstp, nchunks=1)
    ge = _edge3(ge, gb, gc, eWa, eb1, eW2, eb2, eW3, eb3, bm=512)
    a0, a1 = _jnp_scatter2(ge, g2m_dstp, NMp, 1, zeros128)
    v1a, v1b, b1, v2, b2, v3, b3 = _split_node_w(p['enc_node'])
    mx, Pb, Pc = _node3(mx, a0, a1, v1a, b1, v1b, v2, b2, v3, b3,
                        proj=(pe[0][1], pe[0][2]), bm=736)

    for i in range(9):
        wa, _, _, b1e, w2e, b2e, w3e, b3e = pe[i]
        gb, gc = _sc_gather2(Pb, mm_srcp, Pc, mm_dstp, nchunks=9)
        me = _edge3(me, gb, gc, wa, b1e, w2e, b2e, w3e, b3e, bm=512)
        a0, a1 = _jnp_scatter2(me, mm_dstp, NMp, 9, zeros128)
        v1a, v1b, b1, v2, b2, v3, b3 = pn[i]
        nxt = (pe[i + 1][1], pe[i + 1][2]) if i < 8 else (dWb, dWb)
        mx, Pb, Pc = _node3(mx, a0, a1, v1a, b1, v1b, v2, b2, v3, b3,
                            proj=nxt, bm=736)

    gb, gc = _sc_gather2(Pb, m2g_srcp, Pc_dec, m2g_dstp, nchunks=1)
    de = _edge3(de, gb, gc, dWa, db1, dW2, db2, dW3, db3, bm=512)
    a0, a1 = _jnp_scatter2(de, m2g_dstp, NGp, 1, zeros128)
    v1a, v1b, b1, v2, b2, v3, b3 = _split_node_w(p['dec_node'])
    gx = _node3(gx, a0, a1, v1a, b1, v1b, v2, b2, v3, b3, proj=None, bm=NGp)

    out = _out3(gx, p['out'])
    return out[:288, :78][None]
